# baseline (speedup 1.0000x reference)
import jax
import jax.numpy as jnp
from jax import lax
from jax.experimental import pallas as pl
from jax.experimental.pallas import tpu as pltpu
from jax.experimental.pallas import tpu_sc as plsc

NUM_BANDS_K = 200
TOP_K_K = 30
D = 128
NROWS_IN = 16 * 128 * NUM_BANDS_K
NC, NS = 2, 16
NW = NC * NS
CHUNK = 128
CH_PER_W = 6
NCHUNKS = 480
SC_BLKS = NW * CH_PER_W
TC_BLKS = NCHUNKS - SC_BLKS
NBUF = 4


def _sc_body(tab_hbm, sel_hbm, out_hbm, sel_v, idx_v, b0, b1, b2, b3,
             rsem, wsem):
    bufs = (b0, b1, b2, b3)
    c = lax.axis_index("c")
    s = lax.axis_index("s")
    wid = s * NC + c

    pltpu.sync_copy(sel_hbm, sel_v)
    ch_base = wid * CH_PER_W

    iotas = [lax.iota(jnp.int32, 16) + (16 * k) for k in range(8)]

    def build_idx(g):
        u = wid * CH_PER_W + g
        bimg = lax.div(u, TOP_K_K)
        sband = lax.rem(u, TOP_K_K)
        selv = plsc.load_gather(
            sel_v, [lax.broadcast_in_dim(sband, (16,), ())])
        base = bimg * (128 * NUM_BANDS_K)
        for k in range(8):
            idx_v[pl.ds(g * CHUNK + 16 * k, 16)] = (
                base + iotas[k] * NUM_BANDS_K + selv)

    def gather_start(g, b):
        pltpu.async_copy(tab_hbm.at[idx_v.at[pl.ds(g * CHUNK, CHUNK)]],
                         bufs[b], rsem.at[b])

    def gather_wait(b):
        pltpu.make_async_copy(tab_hbm.at[idx_v.at[pl.ds(0, CHUNK)]],
                              bufs[b], rsem.at[b]).wait()

    def wr_start(g, b):
        pltpu.async_copy(bufs[b], out_hbm.at[ch_base + g], wsem.at[b])

    def wr_wait(b):
        pltpu.make_async_copy(bufs[b], out_hbm.at[0], wsem.at[b]).wait()

    LOOKAHEAD = 3
    for g in range(CH_PER_W):
        build_idx(g)
        if g < LOOKAHEAD:
            gather_start(g, g % NBUF)
    for g in range(CH_PER_W):
        b = g % NBUF
        gather_wait(b)
        wr_start(g, b)
        nxt = g + LOOKAHEAD
        if nxt < CH_PER_W:
            nb = nxt % NBUF
            if nxt >= NBUF:
                wr_wait(nb)
            gather_start(nxt, nb)
    for g in range(CH_PER_W - NBUF, CH_PER_W):
        wr_wait(g % NBUF)


@jax.jit
def kernel(x, selected_bands):
    table = jnp.transpose(x, (0, 1, 3, 2)).reshape(NROWS_IN, D)
    sel = selected_bands.astype(jnp.int32)

    mesh = plsc.VectorSubcoreMesh(
        core_axis_name="c", subcore_axis_name="s", num_cores=NC,
        num_subcores=NS)
    fn = pl.kernel(
        _sc_body,
        out_type=jax.ShapeDtypeStruct((SC_BLKS, CHUNK, D), jnp.float32),
        mesh=mesh,
        compiler_params=pltpu.CompilerParams(needs_layout_passes=False),
        scratch_types=[
            pltpu.VMEM((TOP_K_K,), jnp.int32),
            pltpu.VMEM((CH_PER_W * CHUNK,), jnp.int32),
            pltpu.VMEM((CHUNK, D), jnp.float32),
            pltpu.VMEM((CHUNK, D), jnp.float32),
            pltpu.VMEM((CHUNK, D), jnp.float32),
            pltpu.VMEM((CHUNK, D), jnp.float32),
            pltpu.SemaphoreType.DMA((NBUF,)),
            pltpu.SemaphoreType.DMA((NBUF,)),
        ],
    )
    sc_out = fn(table, sel)

    xt = jnp.transpose(x, (0, 1, 3, 2))

    def _tc_body(sel_ref, x_any, o_any, vbuf, rsem, wsem):
        i = pl.program_id(0)
        nsteps = pl.num_programs(0)

        def rd(step):
            u = SC_BLKS + step
            b = u // TOP_K_K
            sband = sel_ref[u % TOP_K_K]
            slot = lax.rem(step, 4)
            return pltpu.make_async_copy(
                x_any.at[b, :, sband, :], vbuf.at[slot], rsem.at[slot])

        def wr(step):
            slot = lax.rem(step, 4)
            return pltpu.make_async_copy(
                vbuf.at[slot], o_any.at[step], wsem.at[slot])

        @pl.when(i == 0)
        def _():
            rd(0).start()
            rd(1).start()
            rd(2).start()

        rd(i).wait()
        wr(i).start()

        @pl.when(i + 3 < nsteps)
        def _():
            @pl.when(i >= 1)
            def _():
                wr(i - 1).wait()

            rd(i + 3).start()

        @pl.when(i == nsteps - 1)
        def _():
            wr(i - 3).wait()
            wr(i - 2).wait()
            wr(i - 1).wait()
            wr(i).wait()

    tc_out = pl.pallas_call(
        _tc_body,
        grid_spec=pltpu.PrefetchScalarGridSpec(
            num_scalar_prefetch=1,
            grid=(TC_BLKS,),
            in_specs=[pl.BlockSpec(memory_space=pl.ANY)],
            out_specs=pl.BlockSpec(memory_space=pl.ANY),
            scratch_shapes=[
                pltpu.VMEM((4, CHUNK, D), jnp.float32),
                pltpu.SemaphoreType.DMA((4,)),
                pltpu.SemaphoreType.DMA((4,)),
            ],
        ),
        out_shape=jax.ShapeDtypeStruct((TC_BLKS, CHUNK, D), jnp.float32),
    )(sel, xt)

    out = jnp.concatenate([sc_out, tc_out], axis=0)
    out4 = out.reshape(16, TOP_K_K, 128, 128)
    return jnp.transpose(out4, (0, 2, 3, 1))


# speedup vs baseline: 3.5894x; 3.5894x over previous
import jax
import jax.numpy as jnp
from jax import lax
from jax.experimental import pallas as pl
from jax.experimental.pallas import tpu as pltpu
from jax.experimental.pallas import tpu_sc as plsc

NUM_BANDS_K = 200
TOP_K_K = 30
D = 128
NROWS_IN = 16 * 128 * NUM_BANDS_K
NC, NS = 2, 16
NW = NC * NS
CHUNK = 128
CH_PER_W = 15
NCHUNKS = NW * CH_PER_W
NBUF = 6


def _sc_body(tab_hbm, sel_hbm, out_hbm, sel_v, idx_v, b0, b1, b2, b3, b4,
             b5, rsem, wsem):
    bufs = (b0, b1, b2, b3, b4, b5)
    c = lax.axis_index("c")
    s = lax.axis_index("s")
    wid = s * NC + c

    pltpu.sync_copy(sel_hbm, sel_v)
    ch_base = wid * CH_PER_W

    iotas = [lax.iota(jnp.int32, 16) + (16 * k) for k in range(8)]

    def build_idx(g):
        u = wid * CH_PER_W + g
        bimg = lax.div(u, TOP_K_K)
        sband = lax.rem(u, TOP_K_K)
        selv = plsc.load_gather(
            sel_v, [lax.broadcast_in_dim(sband, (16,), ())])
        base = bimg * (128 * NUM_BANDS_K)
        for k in range(8):
            idx_v[pl.ds(g * CHUNK + 16 * k, 16)] = (
                base + iotas[k] * NUM_BANDS_K + selv)

    def gather_start(g, b):
        pltpu.async_copy(tab_hbm.at[idx_v.at[pl.ds(g * CHUNK, CHUNK)]],
                         bufs[b], rsem.at[b])

    def gather_wait(b):
        pltpu.make_async_copy(tab_hbm.at[idx_v.at[pl.ds(0, CHUNK)]],
                              bufs[b], rsem.at[b]).wait()

    def wr_start(g, b):
        pltpu.async_copy(bufs[b], out_hbm.at[ch_base + g], wsem.at[b])

    def wr_wait(b):
        pltpu.make_async_copy(bufs[b], out_hbm.at[0], wsem.at[b]).wait()

    LOOKAHEAD = 5
    for g in range(CH_PER_W):
        build_idx(g)
        if g < LOOKAHEAD:
            gather_start(g, g % NBUF)
    for g in range(CH_PER_W):
        b = g % NBUF
        gather_wait(b)
        wr_start(g, b)
        nxt = g + LOOKAHEAD
        if nxt < CH_PER_W:
            nb = nxt % NBUF
            if nxt >= NBUF:
                wr_wait(nb)
            gather_start(nxt, nb)
    for g in range(CH_PER_W - NBUF, CH_PER_W):
        wr_wait(g % NBUF)


@jax.jit
def kernel(x, selected_bands):
    table = jnp.transpose(x, (0, 1, 3, 2)).reshape(NROWS_IN, D)
    sel = selected_bands.astype(jnp.int32)

    mesh = plsc.VectorSubcoreMesh(
        core_axis_name="c", subcore_axis_name="s", num_cores=NC,
        num_subcores=NS)
    fn = pl.kernel(
        _sc_body,
        out_type=jax.ShapeDtypeStruct((NCHUNKS, CHUNK, D), jnp.float32),
        mesh=mesh,
        compiler_params=pltpu.CompilerParams(needs_layout_passes=False),
        scratch_types=[
            pltpu.VMEM((TOP_K_K,), jnp.int32),
            pltpu.VMEM((CH_PER_W * CHUNK,), jnp.int32),
            pltpu.VMEM((CHUNK, D), jnp.float32),
            pltpu.VMEM((CHUNK, D), jnp.float32),
            pltpu.VMEM((CHUNK, D), jnp.float32),
            pltpu.VMEM((CHUNK, D), jnp.float32),
            pltpu.VMEM((CHUNK, D), jnp.float32),
            pltpu.VMEM((CHUNK, D), jnp.float32),
            pltpu.SemaphoreType.DMA((NBUF,)),
            pltpu.SemaphoreType.DMA((NBUF,)),
        ],
    )
    out = fn(table, sel)
    out4 = out.reshape(16, TOP_K_K, 128, 128)
    return jnp.transpose(out4, (0, 2, 3, 1))


# speedup vs baseline: 3.6006x; 1.0031x over previous
import jax
import jax.numpy as jnp
from jax import lax
from jax.experimental import pallas as pl
from jax.experimental.pallas import tpu as pltpu
from jax.experimental.pallas import tpu_sc as plsc

NUM_BANDS_K = 200
TOP_K_K = 30
D = 128
NROWS_IN = 16 * 128 * NUM_BANDS_K
NC, NS = 2, 16
NW = NC * NS
CHUNK = 128
CH_PER_W = 15
NCHUNKS = NW * CH_PER_W
NBUF = 5
LOOKAHEAD = 4


def _sc_body(tab_hbm, sel_hbm, out_hbm, sel_v, idx_v, b0, b1, b2, b3, b4,
             rsem, wsem):
    bufs = (b0, b1, b2, b3, b4)
    c = lax.axis_index("c")
    s = lax.axis_index("s")
    wid = s * NC + c

    pltpu.sync_copy(sel_hbm, sel_v)
    ch_base = wid * CH_PER_W

    @plsc.parallel_loop(0, CH_PER_W * 8, 1, unroll=4)
    def _(v):
        g = lax.div(v, 8)
        k = lax.rem(v, 8)
        u = wid * CH_PER_W + g
        bimg = lax.div(u, TOP_K_K)
        sband = lax.rem(u, TOP_K_K)
        selv = plsc.load_gather(
            sel_v, [lax.broadcast_in_dim(sband, (16,), ())])
        j = lax.iota(jnp.int32, 16) + k * 16
        idx_v[pl.ds(v * 16, 16)] = (
            bimg * (128 * NUM_BANDS_K) + j * NUM_BANDS_K + selv)

    def gather_start(g, b):
        pltpu.async_copy(tab_hbm.at[idx_v.at[pl.ds(g * CHUNK, CHUNK)]],
                         bufs[b], rsem.at[b])

    def gather_wait(b):
        pltpu.make_async_copy(tab_hbm.at[idx_v.at[pl.ds(0, CHUNK)]],
                              bufs[b], rsem.at[b]).wait()

    def wr_start(g, b):
        pltpu.async_copy(bufs[b], out_hbm.at[ch_base + g], wsem.at[b])

    def wr_wait(b):
        pltpu.make_async_copy(bufs[b], out_hbm.at[0], wsem.at[b]).wait()

    for g in range(LOOKAHEAD):
        gather_start(g, g)

    def quint(t, carry):
        for b5 in range(NBUF):
            g = NBUF * t + b5
            nxt = g + LOOKAHEAD
            nb = (b5 + LOOKAHEAD) % NBUF

            @pl.when(nxt < CH_PER_W)
            def _():
                @pl.when(nxt >= NBUF)
                def _():
                    wr_wait(nb)

                gather_start(nxt, nb)

            gather_wait(b5)
            wr_start(g, b5)
        return carry

    lax.fori_loop(0, CH_PER_W // NBUF, quint, 0, unroll=False)
    for b in range(NBUF):
        wr_wait(b)


@jax.jit
def kernel(x, selected_bands):
    table = jnp.transpose(x, (0, 1, 3, 2)).reshape(NROWS_IN, D)
    sel = selected_bands.astype(jnp.int32)

    mesh = plsc.VectorSubcoreMesh(
        core_axis_name="c", subcore_axis_name="s", num_cores=NC,
        num_subcores=NS)
    fn = pl.kernel(
        _sc_body,
        out_type=jax.ShapeDtypeStruct((NCHUNKS, CHUNK, D), jnp.float32),
        mesh=mesh,
        compiler_params=pltpu.CompilerParams(needs_layout_passes=False),
        scratch_types=[
            pltpu.VMEM((TOP_K_K,), jnp.int32),
            pltpu.VMEM((CH_PER_W * CHUNK,), jnp.int32),
            pltpu.VMEM((CHUNK, D), jnp.float32),
            pltpu.VMEM((CHUNK, D), jnp.float32),
            pltpu.VMEM((CHUNK, D), jnp.float32),
            pltpu.VMEM((CHUNK, D), jnp.float32),
            pltpu.VMEM((CHUNK, D), jnp.float32),
            pltpu.SemaphoreType.DMA((NBUF,)),
            pltpu.SemaphoreType.DMA((NBUF,)),
        ],
    )
    out = fn(table, sel)
    out4 = out.reshape(16, TOP_K_K, 128, 128)
    return jnp.transpose(out4, (0, 2, 3, 1))
